# baseline (speedup 1.0000x reference)
import functools

import jax
import jax.numpy as jnp
from jax import lax
from jax.experimental import pallas as pl
from jax.experimental.pallas import tpu as pltpu
from jax.experimental.pallas import tpu_sc as plsc

N = 10000
D = 128
DH = 64
E = 320000
HD = 256
NCLS = 10
R = 64

NC = 2
NS = 16
CH = 128
EPT = 20480
NCHK = EPT // CH
NBUF = 4
NCHKA = NCHK + NBUF
WIN = 40
NWIN = NCHK // WIN
NTRASH = 112
RPT = 632
NPAD = NS * RPT
SPT = 625

ROWS = 1000
GRID = N // ROWS


def _sc_gather_scatter(h, srcr, dstr, zrows):
    mesh = plsc.VectorSubcoreMesh(core_axis_name="c", subcore_axis_name="s")

    @functools.partial(
        pl.kernel,
        out_type=jax.ShapeDtypeStruct((NPAD, D), jnp.float32),
        mesh=mesh,
        compiler_params=pltpu.CompilerParams(use_tc_tiling_on_sc=False),
        scratch_types=[
            pltpu.VMEM((WIN + NBUF, CH), jnp.int32),
            pltpu.VMEM((WIN, CH), jnp.int32),
            pltpu.VMEM((NBUF, CH, DH), jnp.float32),
            pltpu.VMEM_SHARED((N, DH), jnp.float32),
            pltpu.VMEM_SHARED((NPAD, DH), jnp.float32),
        ] + [pltpu.SemaphoreType.DMA] * (2 * NBUF),
    )
    def g(h_hbm, src_hbm, dst_hbm, z_hbm, out_hbm,
          srcw, dstw, buf, hsp, agg, *sems):
        c = lax.axis_index("c")
        s = lax.axis_index("s")
        gsem = sems[:NBUF]
        ssem = sems[NBUF:]
        pltpu.sync_copy(h_hbm.at[pl.ds(s * SPT, SPT), pl.ds(c * DH, DH)],
                        hsp.at[pl.ds(s * SPT, SPT)])
        pltpu.sync_copy(z_hbm.at[:, pl.ds(0, DH)], agg.at[pl.ds(s * RPT, RPT)])
        plsc.subcore_barrier()

        def window(w, carry):
            pltpu.sync_copy(src_hbm.at[s, pl.ds(w * WIN, WIN + NBUF)], srcw)
            pltpu.sync_copy(dst_hbm.at[s, pl.ds(w * WIN, WIN)], dstw)
            for b in range(NBUF):
                pltpu.async_copy(hsp.at[srcw.at[b]], buf.at[b], gsem[b])

            def quad(i, c2):
                ci = NBUF * i
                for b in range(NBUF):
                    pltpu.make_async_copy(hsp.at[srcw.at[ci + b]], buf.at[b],
                                          gsem[b]).wait()
                    pltpu.async_copy(buf.at[b], agg.at[dstw.at[ci + b]],
                                     ssem[b], add=True)
                for b in range(NBUF):
                    pltpu.make_async_copy(buf.at[b], agg.at[dstw.at[0]],
                                          ssem[b]).wait()
                    pltpu.async_copy(hsp.at[srcw.at[ci + NBUF + b]],
                                     buf.at[b], gsem[b])
                return c2

            lax.fori_loop(0, WIN // NBUF, quad, 0)
            for b in range(NBUF):
                pltpu.make_async_copy(hsp.at[srcw.at[0]], buf.at[b],
                                      gsem[b]).wait()
            return carry

        lax.fori_loop(0, NWIN, window, 0)
        plsc.subcore_barrier()
        pltpu.sync_copy(agg.at[pl.ds(s * RPT, RPT)],
                        out_hbm.at[pl.ds(s * RPT, RPT), pl.ds(c * DH, DH)])

    return g(h, srcr, dstr, zrows)


def _embed_body(x_ref, emb_ref, out_ref):
    xv = x_ref[0, 0, :]
    oh = (xv[:, None] == lax.broadcasted_iota(jnp.int32, (ROWS, 16), 1))
    out_ref[...] = jnp.dot(oh.astype(jnp.float32), emb_ref[...],
                           preferred_element_type=jnp.float32)


def _tc_embed(x3, emb16):
    return pl.pallas_call(
        _embed_body,
        grid=(GRID,),
        in_specs=[pl.BlockSpec((1, 1, ROWS), lambda i: (i, 0, 0)),
                  pl.BlockSpec((16, D), lambda i: (0, 0))],
        out_specs=pl.BlockSpec((ROWS, D), lambda i: (i, 0)),
        out_shape=jax.ShapeDtypeStruct((N, D), jnp.float32),
    )(x3, emb16)


def _linear_body(h_ref, w_ref, b_ref, out_ref):
    out_ref[...] = jnp.dot(h_ref[...], w_ref[...],
                           preferred_element_type=jnp.float32) + b_ref[...]


def _tc_linear(h, w, b2d):
    return pl.pallas_call(
        _linear_body,
        grid=(GRID,),
        in_specs=[pl.BlockSpec((ROWS, D), lambda i: (i, 0)),
                  pl.BlockSpec((D, D), lambda i: (0, 0)),
                  pl.BlockSpec((1, D), lambda i: (0, 0))],
        out_specs=pl.BlockSpec((ROWS, D), lambda i: (i, 0)),
        out_shape=jax.ShapeDtypeStruct((N, D), jnp.float32),
    )(h, w, b2d)


def _update_body(agg_ref, p_ref, w_ref, out_ref):
    out_ref[...] = jnp.maximum(
        jnp.dot(agg_ref[...], w_ref[...], preferred_element_type=jnp.float32)
        + p_ref[...], 0.0)


def _tc_update(agg, p, w):
    return pl.pallas_call(
        _update_body,
        grid=(GRID,),
        in_specs=[pl.BlockSpec((ROWS, D), lambda i: (i, 0)),
                  pl.BlockSpec((ROWS, D), lambda i: (i, 0)),
                  pl.BlockSpec((D, D), lambda i: (0, 0))],
        out_specs=pl.BlockSpec((ROWS, D), lambda i: (i, 0)),
        out_shape=jax.ShapeDtypeStruct((N, D), jnp.float32),
    )(agg, p, w)


def _readout_body(h_ref, r_ref, w1_ref, b1_ref, w2_ref, b2_ref,
                  w3_ref, b3_ref, out_ref, acc_ref):
    i = pl.program_id(0)

    @pl.when(i == 0)
    def _():
        acc_ref[...] = jnp.zeros_like(acc_ref)

    roots = r_ref[0, 0, :]
    col = lax.broadcasted_iota(jnp.int32, (R, ROWS), 1) + i * ROWS
    oh = (roots[:, None] == col).astype(jnp.float32)
    acc_ref[...] += jnp.dot(oh, h_ref[...], preferred_element_type=jnp.float32)

    @pl.when(i == pl.num_programs(0) - 1)
    def _():
        z = jnp.maximum(
            jnp.dot(acc_ref[...], w1_ref[...],
                    preferred_element_type=jnp.float32) + b1_ref[...], 0.0)
        z = jnp.maximum(
            jnp.dot(z, w2_ref[...],
                    preferred_element_type=jnp.float32) + b2_ref[...], 0.0)
        out_ref[...] = jnp.dot(z, w3_ref[...],
                               preferred_element_type=jnp.float32) + b3_ref[...]


def _tc_readout(h, roots3, w1, b1_2d, w2, b2_2d, w3p, b3p):
    return pl.pallas_call(
        _readout_body,
        grid=(GRID,),
        in_specs=[pl.BlockSpec((ROWS, D), lambda i: (i, 0)),
                  pl.BlockSpec((1, 1, R), lambda i: (0, 0, 0)),
                  pl.BlockSpec((D, HD), lambda i: (0, 0)),
                  pl.BlockSpec((1, HD), lambda i: (0, 0)),
                  pl.BlockSpec((HD, HD), lambda i: (0, 0)),
                  pl.BlockSpec((1, HD), lambda i: (0, 0)),
                  pl.BlockSpec((HD, D), lambda i: (0, 0)),
                  pl.BlockSpec((1, D), lambda i: (0, 0))],
        out_specs=pl.BlockSpec((R, D), lambda i: (0, 0)),
        out_shape=jax.ShapeDtypeStruct((R, D), jnp.float32),
        scratch_shapes=[pltpu.VMEM((R, D), jnp.float32)],
    )(h, roots3, w1, b1_2d, w2, b2_2d, w3p, b3p)


def kernel(x, edge_index, root_ids, Emb, W_msg, W_upd, b_upd,
           W1, b1, W2, b2, W3, b3):
    x3 = x.astype(jnp.int32).reshape(GRID, 1, ROWS)
    src = edge_index[0].astype(jnp.int32).reshape(NS, E // NS)
    dst = edge_index[1].astype(jnp.int32).reshape(NS, E // NS)
    padw = EPT - E // NS
    dummy = NBUF * CH
    src_pad = jnp.zeros((NS, padw + dummy), jnp.int32)
    dst_pad = jnp.broadcast_to(
        N + (jnp.arange(padw + dummy, dtype=jnp.int32) % NTRASH),
        (NS, padw + dummy))
    srcr = jnp.concatenate([src, src_pad], axis=1).reshape(NS, NCHKA, CH)
    dstr = jnp.concatenate([dst, dst_pad], axis=1).reshape(NS, NCHKA, CH)
    zrows = jnp.zeros((RPT, D), jnp.float32)
    emb16 = jnp.zeros((16, D), jnp.float32).at[:Emb.shape[0]].set(Emb)
    bu2d = b_upd.reshape(1, D)

    h = _tc_embed(x3, emb16)
    for _ in range(2):
        p = _tc_linear(h, W_upd, bu2d)
        agg = _sc_gather_scatter(h, srcr, dstr, zrows)
        h = _tc_update(agg, p, W_msg)

    roots3 = root_ids.astype(jnp.int32).reshape(1, 1, R)
    w3p = jnp.zeros((HD, D), jnp.float32).at[:, :NCLS].set(W3)
    b3p = jnp.zeros((1, D), jnp.float32).at[0, :NCLS].set(b3)
    out = _tc_readout(h, roots3, W1, b1.reshape(1, HD), W2,
                      b2.reshape(1, HD), w3p, b3p)
    return out[:, :NCLS]


# speedup vs baseline: 1.1186x; 1.1186x over previous
import functools

import jax
import jax.numpy as jnp
from jax import lax
from jax.experimental import pallas as pl
from jax.experimental.pallas import tpu as pltpu
from jax.experimental.pallas import tpu_sc as plsc

N = 10000
D = 128
DH = 64
E = 320000
HD = 256
NCLS = 10
R = 64

NC = 2
NS = 16
CH = 256
EPT = 20480
NCHK = EPT // CH
NCHKA = NCHK + 2
WIN = 20
NWIN = NCHK // WIN
NTRASH = 112
RPT = 632
NPAD = NS * RPT
SPT = 625

ROWS = 1000
GRID = N // ROWS


def _sc_gather_scatter(h, srcr, dstr, zrows):
    mesh = plsc.VectorSubcoreMesh(core_axis_name="c", subcore_axis_name="s")

    @functools.partial(
        pl.kernel,
        out_type=jax.ShapeDtypeStruct((NPAD, D), jnp.float32),
        mesh=mesh,
        compiler_params=pltpu.CompilerParams(use_tc_tiling_on_sc=False),
        scratch_types=[
            pltpu.VMEM((WIN + 2, CH), jnp.int32),
            pltpu.VMEM((WIN, CH), jnp.int32),
            pltpu.VMEM((2, CH, DH), jnp.float32),
            pltpu.VMEM_SHARED((N, DH), jnp.float32),
            pltpu.VMEM_SHARED((NPAD, DH), jnp.float32),
            pltpu.SemaphoreType.DMA,
            pltpu.SemaphoreType.DMA,
        ],
    )
    def g(h_hbm, src_hbm, dst_hbm, z_hbm, out_hbm,
          srcw, dstw, buf, hsp, agg, sem0, sem1):
        c = lax.axis_index("c")
        s = lax.axis_index("s")
        pltpu.sync_copy(h_hbm.at[pl.ds(s * SPT, SPT), pl.ds(c * DH, DH)],
                        hsp.at[pl.ds(s * SPT, SPT)])
        pltpu.sync_copy(z_hbm.at[:, pl.ds(0, DH)], agg.at[pl.ds(s * RPT, RPT)])
        plsc.subcore_barrier()

        sems = (sem0, sem1)

        def window(w, carry):
            pltpu.sync_copy(src_hbm.at[s, pl.ds(w * WIN, WIN + 2)], srcw)
            pltpu.sync_copy(dst_hbm.at[s, pl.ds(w * WIN, WIN)], dstw)
            pltpu.async_copy(hsp.at[srcw.at[0]], buf.at[0], sem0)
            pltpu.async_copy(hsp.at[srcw.at[1]], buf.at[1], sem1)

            def pair(i, c2):
                ci = 2 * i
                for b in (0, 1):
                    cj = ci + b
                    pltpu.make_async_copy(hsp.at[srcw.at[cj]], buf.at[b],
                                          sems[b]).wait()
                    pltpu.sync_copy(buf.at[b], agg.at[dstw.at[cj]], add=True)
                    pltpu.async_copy(hsp.at[srcw.at[cj + 2]], buf.at[b],
                                     sems[b])
                return c2

            lax.fori_loop(0, WIN // 2, pair, 0)
            pltpu.make_async_copy(hsp.at[srcw.at[0]], buf.at[0], sem0).wait()
            pltpu.make_async_copy(hsp.at[srcw.at[0]], buf.at[1], sem1).wait()
            return carry

        lax.fori_loop(0, NWIN, window, 0)
        plsc.subcore_barrier()
        pltpu.sync_copy(agg.at[pl.ds(s * RPT, RPT)],
                        out_hbm.at[pl.ds(s * RPT, RPT), pl.ds(c * DH, DH)])

    return g(h, srcr, dstr, zrows)


def _embed_body(x_ref, emb_ref, out_ref):
    xv = x_ref[0, 0, :]
    oh = (xv[:, None] == lax.broadcasted_iota(jnp.int32, (ROWS, 16), 1))
    out_ref[...] = jnp.dot(oh.astype(jnp.float32), emb_ref[...],
                           preferred_element_type=jnp.float32)


def _tc_embed(x3, emb16):
    return pl.pallas_call(
        _embed_body,
        grid=(GRID,),
        in_specs=[pl.BlockSpec((1, 1, ROWS), lambda i: (i, 0, 0)),
                  pl.BlockSpec((16, D), lambda i: (0, 0))],
        out_specs=pl.BlockSpec((ROWS, D), lambda i: (i, 0)),
        out_shape=jax.ShapeDtypeStruct((N, D), jnp.float32),
    )(x3, emb16)


def _linear_body(h_ref, w_ref, b_ref, out_ref):
    out_ref[...] = jnp.dot(h_ref[...], w_ref[...],
                           preferred_element_type=jnp.float32) + b_ref[...]


def _tc_linear(h, w, b2d):
    return pl.pallas_call(
        _linear_body,
        grid=(GRID,),
        in_specs=[pl.BlockSpec((ROWS, D), lambda i: (i, 0)),
                  pl.BlockSpec((D, D), lambda i: (0, 0)),
                  pl.BlockSpec((1, D), lambda i: (0, 0))],
        out_specs=pl.BlockSpec((ROWS, D), lambda i: (i, 0)),
        out_shape=jax.ShapeDtypeStruct((N, D), jnp.float32),
    )(h, w, b2d)


def _update_body(agg_ref, p_ref, w_ref, out_ref):
    out_ref[...] = jnp.maximum(
        jnp.dot(agg_ref[...], w_ref[...], preferred_element_type=jnp.float32)
        + p_ref[...], 0.0)


def _tc_update(agg, p, w):
    return pl.pallas_call(
        _update_body,
        grid=(GRID,),
        in_specs=[pl.BlockSpec((ROWS, D), lambda i: (i, 0)),
                  pl.BlockSpec((ROWS, D), lambda i: (i, 0)),
                  pl.BlockSpec((D, D), lambda i: (0, 0))],
        out_specs=pl.BlockSpec((ROWS, D), lambda i: (i, 0)),
        out_shape=jax.ShapeDtypeStruct((N, D), jnp.float32),
    )(agg, p, w)


def _readout_body(h_ref, r_ref, w1_ref, b1_ref, w2_ref, b2_ref,
                  w3_ref, b3_ref, out_ref, acc_ref):
    i = pl.program_id(0)

    @pl.when(i == 0)
    def _():
        acc_ref[...] = jnp.zeros_like(acc_ref)

    roots = r_ref[0, 0, :]
    col = lax.broadcasted_iota(jnp.int32, (R, ROWS), 1) + i * ROWS
    oh = (roots[:, None] == col).astype(jnp.float32)
    acc_ref[...] += jnp.dot(oh, h_ref[...], preferred_element_type=jnp.float32)

    @pl.when(i == pl.num_programs(0) - 1)
    def _():
        z = jnp.maximum(
            jnp.dot(acc_ref[...], w1_ref[...],
                    preferred_element_type=jnp.float32) + b1_ref[...], 0.0)
        z = jnp.maximum(
            jnp.dot(z, w2_ref[...],
                    preferred_element_type=jnp.float32) + b2_ref[...], 0.0)
        out_ref[...] = jnp.dot(z, w3_ref[...],
                               preferred_element_type=jnp.float32) + b3_ref[...]


def _tc_readout(h, roots3, w1, b1_2d, w2, b2_2d, w3p, b3p):
    return pl.pallas_call(
        _readout_body,
        grid=(GRID,),
        in_specs=[pl.BlockSpec((ROWS, D), lambda i: (i, 0)),
                  pl.BlockSpec((1, 1, R), lambda i: (0, 0, 0)),
                  pl.BlockSpec((D, HD), lambda i: (0, 0)),
                  pl.BlockSpec((1, HD), lambda i: (0, 0)),
                  pl.BlockSpec((HD, HD), lambda i: (0, 0)),
                  pl.BlockSpec((1, HD), lambda i: (0, 0)),
                  pl.BlockSpec((HD, D), lambda i: (0, 0)),
                  pl.BlockSpec((1, D), lambda i: (0, 0))],
        out_specs=pl.BlockSpec((R, D), lambda i: (0, 0)),
        out_shape=jax.ShapeDtypeStruct((R, D), jnp.float32),
        scratch_shapes=[pltpu.VMEM((R, D), jnp.float32)],
    )(h, roots3, w1, b1_2d, w2, b2_2d, w3p, b3p)


def kernel(x, edge_index, root_ids, Emb, W_msg, W_upd, b_upd,
           W1, b1, W2, b2, W3, b3):
    x3 = x.astype(jnp.int32).reshape(GRID, 1, ROWS)
    src = edge_index[0].astype(jnp.int32).reshape(NS, E // NS)
    dst = edge_index[1].astype(jnp.int32).reshape(NS, E // NS)
    padw = EPT - E // NS
    dummy = 2 * CH
    src_pad = jnp.zeros((NS, padw + dummy), jnp.int32)
    dst_pad = jnp.broadcast_to(
        N + (jnp.arange(padw + dummy, dtype=jnp.int32) % NTRASH),
        (NS, padw + dummy))
    srcr = jnp.concatenate([src, src_pad], axis=1).reshape(NS, NCHKA, CH)
    dstr = jnp.concatenate([dst, dst_pad], axis=1).reshape(NS, NCHKA, CH)
    zrows = jnp.zeros((RPT, D), jnp.float32)
    emb16 = jnp.zeros((16, D), jnp.float32).at[:Emb.shape[0]].set(Emb)
    bu2d = b_upd.reshape(1, D)

    h = _tc_embed(x3, emb16)
    for _ in range(2):
        p = _tc_linear(h, W_upd, bu2d)
        agg = _sc_gather_scatter(h, srcr, dstr, zrows)
        h = _tc_update(agg, p, W_msg)

    roots3 = root_ids.astype(jnp.int32).reshape(1, 1, R)
    w3p = jnp.zeros((HD, D), jnp.float32).at[:, :NCLS].set(W3)
    b3p = jnp.zeros((1, D), jnp.float32).at[0, :NCLS].set(b3)
    out = _tc_readout(h, roots3, W1, b1.reshape(1, HD), W2,
                      b2.reshape(1, HD), w3p, b3p)
    return out[:, :NCLS]


# speedup vs baseline: 1.1986x; 1.0715x over previous
import functools

import jax
import jax.numpy as jnp
from jax import lax
from jax.experimental import pallas as pl
from jax.experimental.pallas import tpu as pltpu
from jax.experimental.pallas import tpu_sc as plsc

N = 10000
D = 128
DH = 64
E = 320000
HD = 256
NCLS = 10
R = 64

NC = 2
NS = 16
CH = 128
EPT = 20480
NCHK = EPT // CH
NCHKA = NCHK + 2
WIN = 40
NWIN = NCHK // WIN
NTRASH = 112
RPT = 632
NPAD = NS * RPT
SPT = 625

ROWS = 1000
GRID = N // ROWS


def _sc_gather_scatter(h, srcr, dstr, zrows):
    mesh = plsc.VectorSubcoreMesh(core_axis_name="c", subcore_axis_name="s")

    @functools.partial(
        pl.kernel,
        out_type=jax.ShapeDtypeStruct((NPAD, D), jnp.float32),
        mesh=mesh,
        compiler_params=pltpu.CompilerParams(use_tc_tiling_on_sc=False),
        scratch_types=[
            pltpu.VMEM((2, WIN + 2, CH), jnp.int32),
            pltpu.VMEM((2, WIN, CH), jnp.int32),
            pltpu.VMEM((2, CH, DH), jnp.float32),
            pltpu.VMEM_SHARED((N, DH), jnp.float32),
            pltpu.VMEM_SHARED((NPAD, DH), jnp.float32),
            pltpu.SemaphoreType.DMA,
            pltpu.SemaphoreType.DMA,
            pltpu.SemaphoreType.DMA,
        ],
    )
    def g(h_hbm, src_hbm, dst_hbm, z_hbm, out_hbm,
          srcw, dstw, buf, hsp, agg, sem0, sem1, isem):
        c = lax.axis_index("c")
        s = lax.axis_index("s")
        pltpu.async_copy(src_hbm.at[s, pl.ds(0, WIN + 2)], srcw.at[0], isem)
        pltpu.async_copy(dst_hbm.at[s, pl.ds(0, WIN)], dstw.at[0], isem)
        pltpu.sync_copy(h_hbm.at[pl.ds(s * SPT, SPT), pl.ds(c * DH, DH)],
                        hsp.at[pl.ds(s * SPT, SPT)])
        pltpu.sync_copy(z_hbm.at[:, pl.ds(0, DH)], agg.at[pl.ds(s * RPT, RPT)])
        plsc.subcore_barrier()

        sems = (sem0, sem1)

        def window(w, carry):
            wsl = lax.rem(w, 2)
            pltpu.make_async_copy(src_hbm.at[s, pl.ds(0, WIN + 2)],
                                  srcw.at[wsl], isem).wait()
            pltpu.make_async_copy(dst_hbm.at[s, pl.ds(0, WIN)],
                                  dstw.at[wsl], isem).wait()
            pltpu.async_copy(hsp.at[srcw.at[wsl, 0]], buf.at[0], sem0)
            pltpu.async_copy(hsp.at[srcw.at[wsl, 1]], buf.at[1], sem1)

            @pl.when(w + 1 < NWIN)
            def _():
                pltpu.async_copy(src_hbm.at[s, pl.ds((w + 1) * WIN, WIN + 2)],
                                 srcw.at[1 - wsl], isem)
                pltpu.async_copy(dst_hbm.at[s, pl.ds((w + 1) * WIN, WIN)],
                                 dstw.at[1 - wsl], isem)

            def pair(i, c2):
                ci = 2 * i
                for b in (0, 1):
                    cj = ci + b
                    pltpu.make_async_copy(hsp.at[srcw.at[wsl, cj]], buf.at[b],
                                          sems[b]).wait()
                    pltpu.sync_copy(buf.at[b], agg.at[dstw.at[wsl, cj]],
                                    add=True)
                    pltpu.async_copy(hsp.at[srcw.at[wsl, cj + 2]], buf.at[b],
                                     sems[b])
                return c2

            lax.fori_loop(0, WIN // 2, pair, 0)
            pltpu.make_async_copy(hsp.at[srcw.at[0, 0]], buf.at[0],
                                  sem0).wait()
            pltpu.make_async_copy(hsp.at[srcw.at[0, 0]], buf.at[1],
                                  sem1).wait()
            return carry

        lax.fori_loop(0, NWIN, window, 0)
        plsc.subcore_barrier()
        pltpu.sync_copy(agg.at[pl.ds(s * RPT, RPT)],
                        out_hbm.at[pl.ds(s * RPT, RPT), pl.ds(c * DH, DH)])

    return g(h, srcr, dstr, zrows)


def _embed_body(x_ref, emb_ref, out_ref):
    xv = x_ref[0, 0, :]
    oh = (xv[:, None] == lax.broadcasted_iota(jnp.int32, (ROWS, 16), 1))
    out_ref[...] = jnp.dot(oh.astype(jnp.float32), emb_ref[...],
                           preferred_element_type=jnp.float32)


def _tc_embed(x3, emb16):
    return pl.pallas_call(
        _embed_body,
        grid=(GRID,),
        in_specs=[pl.BlockSpec((1, 1, ROWS), lambda i: (i, 0, 0)),
                  pl.BlockSpec((16, D), lambda i: (0, 0))],
        out_specs=pl.BlockSpec((ROWS, D), lambda i: (i, 0)),
        out_shape=jax.ShapeDtypeStruct((N, D), jnp.float32),
    )(x3, emb16)


def _linear_body(h_ref, w_ref, b_ref, out_ref):
    out_ref[...] = jnp.dot(h_ref[...], w_ref[...],
                           preferred_element_type=jnp.float32) + b_ref[...]


def _tc_linear(h, w, b2d):
    return pl.pallas_call(
        _linear_body,
        grid=(GRID,),
        in_specs=[pl.BlockSpec((ROWS, D), lambda i: (i, 0)),
                  pl.BlockSpec((D, D), lambda i: (0, 0)),
                  pl.BlockSpec((1, D), lambda i: (0, 0))],
        out_specs=pl.BlockSpec((ROWS, D), lambda i: (i, 0)),
        out_shape=jax.ShapeDtypeStruct((N, D), jnp.float32),
    )(h, w, b2d)


def _update_body(agg_ref, p_ref, w_ref, out_ref):
    out_ref[...] = jnp.maximum(
        jnp.dot(agg_ref[...], w_ref[...], preferred_element_type=jnp.float32)
        + p_ref[...], 0.0)


def _tc_update(agg, p, w):
    return pl.pallas_call(
        _update_body,
        grid=(GRID,),
        in_specs=[pl.BlockSpec((ROWS, D), lambda i: (i, 0)),
                  pl.BlockSpec((ROWS, D), lambda i: (i, 0)),
                  pl.BlockSpec((D, D), lambda i: (0, 0))],
        out_specs=pl.BlockSpec((ROWS, D), lambda i: (i, 0)),
        out_shape=jax.ShapeDtypeStruct((N, D), jnp.float32),
    )(agg, p, w)


def _readout_body(h_ref, r_ref, w1_ref, b1_ref, w2_ref, b2_ref,
                  w3_ref, b3_ref, out_ref, acc_ref):
    i = pl.program_id(0)

    @pl.when(i == 0)
    def _():
        acc_ref[...] = jnp.zeros_like(acc_ref)

    roots = r_ref[0, 0, :]
    col = lax.broadcasted_iota(jnp.int32, (R, ROWS), 1) + i * ROWS
    oh = (roots[:, None] == col).astype(jnp.float32)
    acc_ref[...] += jnp.dot(oh, h_ref[...], preferred_element_type=jnp.float32)

    @pl.when(i == pl.num_programs(0) - 1)
    def _():
        z = jnp.maximum(
            jnp.dot(acc_ref[...], w1_ref[...],
                    preferred_element_type=jnp.float32) + b1_ref[...], 0.0)
        z = jnp.maximum(
            jnp.dot(z, w2_ref[...],
                    preferred_element_type=jnp.float32) + b2_ref[...], 0.0)
        out_ref[...] = jnp.dot(z, w3_ref[...],
                               preferred_element_type=jnp.float32) + b3_ref[...]


def _tc_readout(h, roots3, w1, b1_2d, w2, b2_2d, w3p, b3p):
    return pl.pallas_call(
        _readout_body,
        grid=(GRID,),
        in_specs=[pl.BlockSpec((ROWS, D), lambda i: (i, 0)),
                  pl.BlockSpec((1, 1, R), lambda i: (0, 0, 0)),
                  pl.BlockSpec((D, HD), lambda i: (0, 0)),
                  pl.BlockSpec((1, HD), lambda i: (0, 0)),
                  pl.BlockSpec((HD, HD), lambda i: (0, 0)),
                  pl.BlockSpec((1, HD), lambda i: (0, 0)),
                  pl.BlockSpec((HD, D), lambda i: (0, 0)),
                  pl.BlockSpec((1, D), lambda i: (0, 0))],
        out_specs=pl.BlockSpec((R, D), lambda i: (0, 0)),
        out_shape=jax.ShapeDtypeStruct((R, D), jnp.float32),
        scratch_shapes=[pltpu.VMEM((R, D), jnp.float32)],
    )(h, roots3, w1, b1_2d, w2, b2_2d, w3p, b3p)


def kernel(x, edge_index, root_ids, Emb, W_msg, W_upd, b_upd,
           W1, b1, W2, b2, W3, b3):
    x3 = x.astype(jnp.int32).reshape(GRID, 1, ROWS)
    src = edge_index[0].astype(jnp.int32).reshape(NS, E // NS)
    dst = edge_index[1].astype(jnp.int32).reshape(NS, E // NS)
    padw = EPT - E // NS
    dummy = 2 * CH
    src_pad = jnp.zeros((NS, padw + dummy), jnp.int32)
    dst_pad = jnp.broadcast_to(
        N + (jnp.arange(padw + dummy, dtype=jnp.int32) % NTRASH),
        (NS, padw + dummy))
    srcr = jnp.concatenate([src, src_pad], axis=1).reshape(NS, NCHKA, CH)
    dstr = jnp.concatenate([dst, dst_pad], axis=1).reshape(NS, NCHKA, CH)
    zrows = jnp.zeros((RPT, D), jnp.float32)
    emb16 = jnp.zeros((16, D), jnp.float32).at[:Emb.shape[0]].set(Emb)
    bu2d = b_upd.reshape(1, D)

    h = _tc_embed(x3, emb16)
    for _ in range(2):
        p = _tc_linear(h, W_upd, bu2d)
        agg = _sc_gather_scatter(h, srcr, dstr, zrows)
        h = _tc_update(agg, p, W_msg)

    roots3 = root_ids.astype(jnp.int32).reshape(1, 1, R)
    w3p = jnp.zeros((HD, D), jnp.float32).at[:, :NCLS].set(W3)
    b3p = jnp.zeros((1, D), jnp.float32).at[0, :NCLS].set(b3)
    out = _tc_readout(h, roots3, W1, b1.reshape(1, HD), W2,
                      b2.reshape(1, HD), w3p, b3p)
    return out[:, :NCLS]


# speedup vs baseline: 1.2772x; 1.0655x over previous
import functools

import jax
import jax.numpy as jnp
from jax import lax
from jax.experimental import pallas as pl
from jax.experimental.pallas import tpu as pltpu
from jax.experimental.pallas import tpu_sc as plsc

N = 10000
D = 128
DH = 64
E = 320000
HD = 256
NCLS = 10
R = 64

NC = 2
NS = 16
CH = 128
EPT = E // NS
WIN = 52
NWIN = 3
TAIL = EPT - NWIN * WIN * CH
RPT = 632
NPAD = NS * RPT
SPT = 625

ROWS = 1000
GRID = N // ROWS


def _sc_gather_scatter(h, ei, zrows):
    mesh = plsc.VectorSubcoreMesh(core_axis_name="c", subcore_axis_name="s")

    @functools.partial(
        pl.kernel,
        out_type=jax.ShapeDtypeStruct((NPAD, D), jnp.float32),
        mesh=mesh,
        compiler_params=pltpu.CompilerParams(use_tc_tiling_on_sc=False),
        scratch_types=[
            pltpu.VMEM((2, WIN * CH), jnp.int32),
            pltpu.VMEM((2, WIN * CH), jnp.int32),
            pltpu.VMEM((2, CH, DH), jnp.float32),
            pltpu.VMEM_SHARED((N, DH), jnp.float32),
            pltpu.VMEM_SHARED((NPAD, DH), jnp.float32),
            pltpu.SemaphoreType.DMA,
            pltpu.SemaphoreType.DMA,
            pltpu.SemaphoreType.DMA,
        ],
    )
    def g(h_hbm, ei_hbm, z_hbm, out_hbm,
          srcw, dstw, buf, hsp, agg, sem0, sem1, isem):
        c = lax.axis_index("c")
        s = lax.axis_index("s")
        base = s * EPT
        pltpu.async_copy(ei_hbm.at[0, pl.ds(base, WIN * CH)], srcw.at[0],
                         isem)
        pltpu.async_copy(ei_hbm.at[1, pl.ds(base, WIN * CH)], dstw.at[0],
                         isem)
        pltpu.async_copy(h_hbm.at[pl.ds(s * SPT, SPT), pl.ds(c * DH, DH)],
                         hsp.at[pl.ds(s * SPT, SPT)], sem0)
        pltpu.async_copy(z_hbm.at[:, pl.ds(0, DH)],
                         agg.at[pl.ds(s * RPT, RPT)], sem1)
        pltpu.make_async_copy(h_hbm.at[pl.ds(s * SPT, SPT), pl.ds(c * DH, DH)],
                              hsp.at[pl.ds(s * SPT, SPT)], sem0).wait()
        pltpu.make_async_copy(z_hbm.at[:, pl.ds(0, DH)],
                              agg.at[pl.ds(s * RPT, RPT)], sem1).wait()
        plsc.subcore_barrier()

        sems = (sem0, sem1)

        def window(w, carry):
            wsl = lax.rem(w, 2)
            pltpu.make_async_copy(ei_hbm.at[0, pl.ds(base, WIN * CH)],
                                  srcw.at[wsl], isem).wait()
            pltpu.make_async_copy(ei_hbm.at[1, pl.ds(base, WIN * CH)],
                                  dstw.at[wsl], isem).wait()
            pltpu.async_copy(hsp.at[srcw.at[wsl, pl.ds(0, CH)]], buf.at[0],
                             sem0)
            pltpu.async_copy(hsp.at[srcw.at[wsl, pl.ds(CH, CH)]], buf.at[1],
                             sem1)

            @pl.when(w + 1 < NWIN)
            def _():
                nb = base + (w + 1) * WIN * CH
                pltpu.async_copy(ei_hbm.at[0, pl.ds(nb, WIN * CH)],
                                 srcw.at[1 - wsl], isem)
                pltpu.async_copy(ei_hbm.at[1, pl.ds(nb, WIN * CH)],
                                 dstw.at[1 - wsl], isem)

            def pair(i, c2):
                ci = 2 * i
                for b in (0, 1):
                    cj = ci + b
                    pltpu.make_async_copy(
                        hsp.at[srcw.at[wsl, pl.ds(cj * CH, CH)]], buf.at[b],
                        sems[b]).wait()
                    pltpu.sync_copy(buf.at[b],
                                    agg.at[dstw.at[wsl, pl.ds(cj * CH, CH)]],
                                    add=True)
                    pltpu.async_copy(
                        hsp.at[srcw.at[wsl, pl.ds((cj + 2) * CH, CH)]],
                        buf.at[b], sems[b])
                return c2

            lax.fori_loop(0, WIN // 2 - 1, pair, 0)
            for b in (0, 1):
                cj = WIN - 2 + b
                pltpu.make_async_copy(
                    hsp.at[srcw.at[wsl, pl.ds(cj * CH, CH)]], buf.at[b],
                    sems[b]).wait()
                pltpu.sync_copy(buf.at[b],
                                agg.at[dstw.at[wsl, pl.ds(cj * CH, CH)]],
                                add=True)
            return carry

        lax.fori_loop(0, NWIN, window, 0)

        pltpu.sync_copy(ei_hbm.at[0, pl.ds(base + NWIN * WIN * CH, TAIL)],
                        srcw.at[0, pl.ds(0, TAIL)])
        pltpu.sync_copy(ei_hbm.at[1, pl.ds(base + NWIN * WIN * CH, TAIL)],
                        dstw.at[0, pl.ds(0, TAIL)])
        pltpu.sync_copy(hsp.at[srcw.at[0, pl.ds(0, TAIL)]],
                        buf.at[0, pl.ds(0, TAIL)])
        pltpu.sync_copy(buf.at[0, pl.ds(0, TAIL)],
                        agg.at[dstw.at[0, pl.ds(0, TAIL)]], add=True)

        plsc.subcore_barrier()
        pltpu.sync_copy(agg.at[pl.ds(s * RPT, RPT)],
                        out_hbm.at[pl.ds(s * RPT, RPT), pl.ds(c * DH, DH)])

    return g(h, ei, zrows)


def _embed_body(x_ref, emb_ref, out_ref):
    xv = x_ref[0, 0, :]
    oh = (xv[:, None] == lax.broadcasted_iota(jnp.int32, (ROWS, 16), 1))
    out_ref[...] = jnp.dot(oh.astype(jnp.float32), emb_ref[...],
                           preferred_element_type=jnp.float32)


def _tc_embed(x3, emb16):
    return pl.pallas_call(
        _embed_body,
        grid=(GRID,),
        in_specs=[pl.BlockSpec((1, 1, ROWS), lambda i: (i, 0, 0)),
                  pl.BlockSpec((16, D), lambda i: (0, 0))],
        out_specs=pl.BlockSpec((ROWS, D), lambda i: (i, 0)),
        out_shape=jax.ShapeDtypeStruct((N, D), jnp.float32),
    )(x3, emb16)


def _linear_body(h_ref, w_ref, b_ref, out_ref):
    out_ref[...] = jnp.dot(h_ref[...], w_ref[...],
                           preferred_element_type=jnp.float32) + b_ref[...]


def _tc_linear(h, w, b2d):
    return pl.pallas_call(
        _linear_body,
        grid=(GRID,),
        in_specs=[pl.BlockSpec((ROWS, D), lambda i: (i, 0)),
                  pl.BlockSpec((D, D), lambda i: (0, 0)),
                  pl.BlockSpec((1, D), lambda i: (0, 0))],
        out_specs=pl.BlockSpec((ROWS, D), lambda i: (i, 0)),
        out_shape=jax.ShapeDtypeStruct((N, D), jnp.float32),
    )(h, w, b2d)


def _update_body(agg_ref, p_ref, w_ref, out_ref):
    out_ref[...] = jnp.maximum(
        jnp.dot(agg_ref[...], w_ref[...], preferred_element_type=jnp.float32)
        + p_ref[...], 0.0)


def _tc_update(agg, p, w):
    return pl.pallas_call(
        _update_body,
        grid=(GRID,),
        in_specs=[pl.BlockSpec((ROWS, D), lambda i: (i, 0)),
                  pl.BlockSpec((ROWS, D), lambda i: (i, 0)),
                  pl.BlockSpec((D, D), lambda i: (0, 0))],
        out_specs=pl.BlockSpec((ROWS, D), lambda i: (i, 0)),
        out_shape=jax.ShapeDtypeStruct((N, D), jnp.float32),
    )(agg, p, w)


def _readout_body(h_ref, r_ref, w1_ref, b1_ref, w2_ref, b2_ref,
                  w3_ref, b3_ref, out_ref, acc_ref):
    i = pl.program_id(0)

    @pl.when(i == 0)
    def _():
        acc_ref[...] = jnp.zeros_like(acc_ref)

    roots = r_ref[0, 0, :]
    col = lax.broadcasted_iota(jnp.int32, (R, ROWS), 1) + i * ROWS
    oh = (roots[:, None] == col).astype(jnp.float32)
    acc_ref[...] += jnp.dot(oh, h_ref[...], preferred_element_type=jnp.float32)

    @pl.when(i == pl.num_programs(0) - 1)
    def _():
        z = jnp.maximum(
            jnp.dot(acc_ref[...], w1_ref[...],
                    preferred_element_type=jnp.float32) + b1_ref[...], 0.0)
        z = jnp.maximum(
            jnp.dot(z, w2_ref[...],
                    preferred_element_type=jnp.float32) + b2_ref[...], 0.0)
        out_ref[...] = jnp.dot(z, w3_ref[...],
                               preferred_element_type=jnp.float32) + b3_ref[...]


def _tc_readout(h, roots3, w1, b1_2d, w2, b2_2d, w3p, b3p):
    return pl.pallas_call(
        _readout_body,
        grid=(GRID,),
        in_specs=[pl.BlockSpec((ROWS, D), lambda i: (i, 0)),
                  pl.BlockSpec((1, 1, R), lambda i: (0, 0, 0)),
                  pl.BlockSpec((D, HD), lambda i: (0, 0)),
                  pl.BlockSpec((1, HD), lambda i: (0, 0)),
                  pl.BlockSpec((HD, HD), lambda i: (0, 0)),
                  pl.BlockSpec((1, HD), lambda i: (0, 0)),
                  pl.BlockSpec((HD, D), lambda i: (0, 0)),
                  pl.BlockSpec((1, D), lambda i: (0, 0))],
        out_specs=pl.BlockSpec((R, D), lambda i: (0, 0)),
        out_shape=jax.ShapeDtypeStruct((R, D), jnp.float32),
        scratch_shapes=[pltpu.VMEM((R, D), jnp.float32)],
    )(h, roots3, w1, b1_2d, w2, b2_2d, w3p, b3p)


def kernel(x, edge_index, root_ids, Emb, W_msg, W_upd, b_upd,
           W1, b1, W2, b2, W3, b3):
    x3 = x.astype(jnp.int32).reshape(GRID, 1, ROWS)
    ei = edge_index.astype(jnp.int32)
    zrows = jnp.zeros((RPT, D), jnp.float32)
    emb16 = jnp.zeros((16, D), jnp.float32).at[:Emb.shape[0]].set(Emb)
    bu2d = b_upd.reshape(1, D)

    h = _tc_embed(x3, emb16)
    for _ in range(2):
        p = _tc_linear(h, W_upd, bu2d)
        agg = _sc_gather_scatter(h, ei, zrows)
        h = _tc_update(agg, p, W_msg)

    roots3 = root_ids.astype(jnp.int32).reshape(1, 1, R)
    w3p = jnp.zeros((HD, D), jnp.float32).at[:, :NCLS].set(W3)
    b3p = jnp.zeros((1, D), jnp.float32).at[0, :NCLS].set(b3)
    out = _tc_readout(h, roots3, W1, b1.reshape(1, HD), W2,
                      b2.reshape(1, HD), w3p, b3p)
    return out[:, :NCLS]
